# baseline (speedup 1.0000x reference)
import dataclasses
import functools

import jax
import jax.numpy as jnp
from jax import lax
from jax.experimental import pallas as pl
from jax.experimental.pallas import tpu as pltpu
from jax.experimental.pallas import tpu_sc as plsc

B = 64
S = 200
P = 128
NC = 2
NS = 16
NW = NC * NS
ROWS_PER_W = B // NW
L = 16
CHUNKS = ((0, 72), (72, 128))


def _sc_forward(X, table, W, bias):
    mesh = plsc.VectorSubcoreMesh(core_axis_name="c", subcore_axis_name="s")
    SW = S * ROWS_PER_W

    cp = pltpu.CompilerParams()
    if "needs_layout_passes" in pltpu.CompilerParams.__dataclass_fields__:
        cp = dataclasses.replace(cp, needs_layout_passes=False)

    @functools.partial(
        pl.kernel,
        mesh=mesh,
        compiler_params=cp,
        out_type=jax.ShapeDtypeStruct((B, L), jnp.float32),
        scratch_types=[
            pltpu.VMEM((SW,), jnp.int32),
            pltpu.VMEM((SW, P), jnp.float32),
            pltpu.VMEM((P,), jnp.float32),
            pltpu.VMEM((L,), jnp.float32),
            pltpu.VMEM((ROWS_PER_W, L), jnp.float32),
            pltpu.SemaphoreType.DMA,
            pltpu.SemaphoreType.DMA,
            pltpu.SemaphoreType.DMA,
        ],
    )
    def k(x_hbm, table_hbm, w_hbm, b_hbm, out_hbm,
          idx_v, rows_v, w_v, b_v, o_v, sem0, sem1, semw):
        wid = lax.axis_index("s") * NC + lax.axis_index("c")
        cw = pltpu.async_copy(w_hbm, w_v, semw)
        cb = pltpu.async_copy(b_hbm, b_v, semw)
        sems = (sem0, sem1)
        idx_copies = [
            pltpu.async_copy(
                x_hbm.at[pl.ds(wid * SW + j * S, S)],
                idx_v.at[pl.ds(j * S, S)],
                sems[j],
            )
            for j in range(ROWS_PER_W)
        ]
        copies = []
        for j in range(ROWS_PER_W):
            idx_copies[j].wait()
            for off, n in CHUNKS:
                copies.append(
                    pltpu.async_copy(
                        table_hbm.at[idx_v.at[pl.ds(j * S + off, n)]],
                        rows_v.at[pl.ds(j * S + off, n)],
                        sems[j],
                    )
                )
        cw.wait()
        cb.wait()
        for j in range(ROWS_PER_W):
            zero = jnp.zeros((L,), jnp.float32)
            acc = (zero,) * (P // L)
            for c_idx, (off, n) in enumerate(CHUNKS):
                copies[j * len(CHUNKS) + c_idx].wait()

                def body(r, a, base=j * S):
                    return tuple(
                        a[c] + rows_v[base + r, pl.ds(c * L, L)]
                        for c in range(P // L)
                    )

                acc = plsc.parallel_loop(
                    off, off + n, unroll=4, carry=acc
                )(body)
            part = zero
            for c in range(P // L):
                part = part + acc[c] * w_v[pl.ds(c * L, L)]
            z = jnp.sum(part) * (1.0 / S)
            zv = jnp.broadcast_to(z, (L,)) + b_v[...]
            o_v[j, :] = 1.0 / (1.0 + jnp.exp(-zv))
        pltpu.sync_copy(o_v, out_hbm.at[pl.ds(wid * ROWS_PER_W, ROWS_PER_W)])

    return k(X.reshape(-1), table, W.reshape(P), jnp.broadcast_to(bias, (L,)))


def kernel(X, table, W, b):
    out = _sc_forward(X, table, W, b)
    return out[:, 0]


# speedup vs baseline: 1.0441x; 1.0441x over previous
import dataclasses
import functools

import jax
import jax.numpy as jnp
from jax import lax
from jax.experimental import pallas as pl
from jax.experimental.pallas import tpu as pltpu
from jax.experimental.pallas import tpu_sc as plsc

B = 64
S = 200
P = 128
NC = 2
NS = 16
NW = NC * NS
ROWS_PER_W = B // NW
L = 16
CHUNKS = ((0, 32), (32, 96), (128, 72))


def _sc_forward(X, table, W, bias):
    mesh = plsc.VectorSubcoreMesh(core_axis_name="c", subcore_axis_name="s")
    SW = S * ROWS_PER_W

    cp = pltpu.CompilerParams()
    if "needs_layout_passes" in pltpu.CompilerParams.__dataclass_fields__:
        cp = dataclasses.replace(cp, needs_layout_passes=False)

    @functools.partial(
        pl.kernel,
        mesh=mesh,
        compiler_params=cp,
        out_type=jax.ShapeDtypeStruct((B,), jnp.float32),
        scratch_types=[
            pltpu.VMEM((SW,), jnp.int32),
            pltpu.VMEM((SW, P), jnp.float32),
            pltpu.VMEM((P,), jnp.float32),
            pltpu.VMEM((L,), jnp.float32),
            pltpu.VMEM((L,), jnp.float32),
            pltpu.VMEM_SHARED((NS, L), jnp.float32),
            pltpu.VMEM((NS, L), jnp.float32),
            pltpu.VMEM((2 * NS,), jnp.float32),
            pltpu.SemaphoreType.DMA,
            pltpu.SemaphoreType.DMA,
            pltpu.SemaphoreType.DMA,
        ],
    )
    def k(x_hbm, table_hbm, w_hbm, b_hbm, out_hbm,
          idx_v, rows_v, w_v, b_v, o_v, shr_v, g_v, g32_v, sem0, sem1, semw):
        c_ax = lax.axis_index("c")
        s_ax = lax.axis_index("s")
        wid = c_ax * NS + s_ax
        cw = pltpu.async_copy(w_hbm, w_v, semw)
        cb = pltpu.async_copy(b_hbm, b_v, semw)
        sems = (sem0, sem1)
        idx_copies = [
            pltpu.async_copy(
                x_hbm.at[pl.ds(wid * SW + j * S, S)],
                idx_v.at[pl.ds(j * S, S)],
                sems[j],
            )
            for j in range(ROWS_PER_W)
        ]
        copies = []
        for j in range(ROWS_PER_W):
            idx_copies[j].wait()
            for off, n in CHUNKS:
                copies.append(
                    pltpu.async_copy(
                        table_hbm.at[idx_v.at[pl.ds(j * S + off, n)]],
                        rows_v.at[pl.ds(j * S + off, n)],
                        sems[j],
                    )
                )
        cw.wait()
        cb.wait()
        sigs = []
        for j in range(ROWS_PER_W):
            zero = jnp.zeros((L,), jnp.float32)
            acc = (zero,) * (P // L)
            for c_idx, (off, n) in enumerate(CHUNKS):
                copies[j * len(CHUNKS) + c_idx].wait()

                def body(r, a, base=j * S):
                    return tuple(
                        a[c] + rows_v[base + r, pl.ds(c * L, L)]
                        for c in range(P // L)
                    )

                acc = plsc.parallel_loop(
                    off, off + n, unroll=4, carry=acc
                )(body)
            part = zero
            for c in range(P // L):
                part = part + acc[c] * w_v[pl.ds(c * L, L)]
            z = jnp.sum(part) * (1.0 / S)
            zv = jnp.broadcast_to(z, (L,)) + b_v[...]
            sigs.append(1.0 / (1.0 + jnp.exp(-zv)))
        io = jax.lax.iota(jnp.int32, L)
        o_v[...] = jnp.where(io == 0, sigs[0], jnp.where(io == 1, sigs[1], 0.0))
        pltpu.sync_copy(o_v, shr_v.at[s_ax])
        plsc.subcore_barrier()

        @pl.when(s_ax == 0)
        def _():
            pltpu.sync_copy(shr_v, g_v)
            for r in range(NS):
                plsc.store_scatter(g32_v, [io + 2 * r], g_v[r, :], mask=io < 2)
            pltpu.sync_copy(g32_v, out_hbm.at[pl.ds(c_ax * 2 * NS, 2 * NS)])

    return k(X.reshape(-1), table, W.reshape(P), jnp.broadcast_to(bias, (L,)))


def kernel(X, table, W, b):
    return _sc_forward(X, table, W, b)
